# baseline (speedup 1.0000x reference)
import functools

import jax
import jax.numpy as jnp
from jax import lax
from jax.experimental import pallas as pl
from jax.experimental.pallas import tpu as pltpu
from jax.experimental.pallas import tpu_sc as plsc

B = 1024
L = 50
H = 128
V = 1000

NC = 2
NS = 16
NW = NC * NS
ROWS_PER_W = B // NW
GROUPS = ROWS_PER_W // 16

BB = 128
PACK_ROWS = 56
TC_BLOCKS = B // BB

MASK_SCORE = -60.0


def _scores_body(last_ref, all_ref, wr_ref, ur_ref, vr_ref, out_ref):
    lm = lax.dot_general(
        last_ref[...], wr_ref[...], (((1,), (1,)), ((), ())),
        preferred_element_type=jnp.float32)
    am = lax.dot_general(
        all_ref[...].reshape(L * BB, H), ur_ref[...],
        (((1,), (1,)), ((), ())),
        preferred_element_type=jnp.float32)
    z = jnp.tanh(am.reshape(L, BB, H) + lm[None, :, :])
    s = lax.dot_general(
        z.reshape(L * BB, H), vr_ref[...], (((1,), (1,)), ((), ())),
        preferred_element_type=jnp.float32)
    s2 = s.reshape(L, BB)
    out_ref[...] = jnp.concatenate(
        [s2, jnp.zeros((PACK_ROWS - L, BB), jnp.float32)], axis=0)


def _tc_scores(last_memory, all_t, Wr, Ur, Vr):
    return pl.pallas_call(
        _scores_body,
        grid=(TC_BLOCKS,),
        in_specs=[
            pl.BlockSpec((BB, H), lambda i: (i, 0)),
            pl.BlockSpec((L, BB, H), lambda i: (0, i, 0)),
            pl.BlockSpec((H, H), lambda i: (0, 0)),
            pl.BlockSpec((H, H), lambda i: (0, 0)),
            pl.BlockSpec((1, H), lambda i: (0, 0)),
        ],
        out_specs=pl.BlockSpec((PACK_ROWS, BB), lambda i: (i, 0)),
        out_shape=jax.ShapeDtypeStruct((TC_BLOCKS * PACK_ROWS, BB),
                                       jnp.float32),
    )(last_memory, all_t, Wr, Ur, Vr)


VQ_STARTS = (0, 256, 512, 768)
VQ_ROWS = 256
VQ_LAST = V - VQ_STARTS[3]


def _sc_scatter_body(s_hbm, seq_hbm, mask_hbm, out_hbm,
                     s_v, seq_v, mask_v, vals_v, acc_v):
    wid = lax.axis_index("s") * NC + lax.axis_index("c")
    t = wid // 4
    q = wid % 4
    vstart = q * VQ_ROWS
    vend = jnp.minimum(vstart + VQ_ROWS, V)
    in_base = t * (BB * L)

    pltpu.sync_copy(s_hbm.at[pl.ds(t * PACK_ROWS, PACK_ROWS), :], s_v)
    pltpu.sync_copy(seq_hbm.at[pl.ds(in_base, BB * L)], seq_v)
    pltpu.sync_copy(mask_hbm.at[pl.ds(in_base, BB * L)], mask_v)

    zeros16 = jnp.zeros((16,), jnp.float32)

    def _zero_rows(i, _):
        for u in range(8):
            acc_v[i, pl.ds(u * 16, 16)] = zeros16
        return 0

    lax.fori_loop(0, VQ_ROWS, _zero_rows, 0)

    lane = lax.iota(jnp.int32, 16)
    for g in range(BB // 16):
        b_idx = lane + g * 16
        lin = b_idx * L
        denom = zeros16
        for l in range(L):
            m = plsc.load_gather(mask_v, [lin + l])
            sv = s_v[l, pl.ds(g * 16, 16)]
            val = jnp.exp(jnp.where(m != 0, MASK_SCORE, sv))
            vals_v[l, pl.ds(g * 16, 16)] = val
            denom = denom + val
        dinv = 1.0 / denom
        for l in range(L):
            col = plsc.load_gather(seq_v, [lin + l])
            val = vals_v[l, pl.ds(g * 16, 16)]
            keep = (col >= vstart) & (col < vend)
            plsc.addupdate_scatter(acc_v, [col - vstart, b_idx],
                                   val * dinv, mask=keep)

    pltpu.sync_copy(acc_v.at[pl.ds(0, VQ_LAST), :],
                    out_hbm.at[pl.ds(vstart, VQ_LAST), pl.ds(t * BB, BB)])

    @pl.when(q < 3)
    def _tail():
        pltpu.sync_copy(
            acc_v.at[pl.ds(VQ_LAST, VQ_ROWS - VQ_LAST), :],
            out_hbm.at[pl.ds(vstart + VQ_LAST, VQ_ROWS - VQ_LAST),
                       pl.ds(t * BB, BB)])


@functools.cache
def _sc_scatter():
    return pl.kernel(
        _sc_scatter_body,
        out_type=jax.ShapeDtypeStruct((V, B), jnp.float32),
        mesh=plsc.VectorSubcoreMesh(core_axis_name="c", subcore_axis_name="s",
                                    num_cores=NC, num_subcores=NS),
        compiler_params=pltpu.CompilerParams(needs_layout_passes=False),
        scratch_types=[
            pltpu.VMEM((PACK_ROWS, BB), jnp.float32),
            pltpu.VMEM((BB * L,), jnp.int32),
            pltpu.VMEM((BB * L,), jnp.int32),
            pltpu.VMEM((L, BB), jnp.float32),
            pltpu.VMEM((VQ_ROWS, BB), jnp.float32),
        ],
    )


def kernel(seq_item, last_memory, all_memory, mask, item_matrix, Wr, Ur, Vr):
    all_t = jnp.transpose(all_memory, (1, 0, 2))
    scores = _tc_scores(last_memory, all_t, Wr, Ur, Vr)
    out_t = _sc_scatter()(scores,
                          seq_item.astype(jnp.int32).reshape(B * L),
                          mask.astype(jnp.int32).reshape(B * L))
    return out_t.T


# speedup vs baseline: 1.3050x; 1.3050x over previous
import functools

import jax
import jax.numpy as jnp
from jax import lax
from jax.experimental import pallas as pl
from jax.experimental.pallas import tpu as pltpu
from jax.experimental.pallas import tpu_sc as plsc

B = 1024
L = 50
H = 128
V = 1000

NC = 2
NS = 16
NW = NC * NS
ROWS_PER_W = B // NW
GROUPS = ROWS_PER_W // 16

BB = 128
PACK_ROWS = 56
TC_BLOCKS = B // BB

MASK_SCORE = -60.0


def _probs_body(last_ref, all_ref, mask_ref, wr_ref, ur_ref, vr_ref, out_ref):
    lm = lax.dot_general(
        last_ref[...], wr_ref[...], (((1,), (1,)), ((), ())),
        preferred_element_type=jnp.float32)
    am = lax.dot_general(
        all_ref[...].reshape(L * BB, H), ur_ref[...],
        (((1,), (1,)), ((), ())),
        preferred_element_type=jnp.float32)
    z = jnp.tanh(am.reshape(L, BB, H) + lm[None, :, :])
    s = lax.dot_general(
        z.reshape(L * BB, H), vr_ref[...], (((1,), (1,)), ((), ())),
        preferred_element_type=jnp.float32)
    s2 = s.reshape(L, BB)
    s2 = jnp.where(mask_ref[...] != 0, MASK_SCORE, s2)
    e = jnp.exp(s2)
    p = e / jnp.sum(e, axis=0, keepdims=True)
    out_ref[...] = jnp.concatenate(
        [p, jnp.zeros((PACK_ROWS - L, BB), jnp.float32)], axis=0)


def _tc_probs(last_memory, all_t, mask_t, Wr, Ur, Vr):
    return pl.pallas_call(
        _probs_body,
        grid=(TC_BLOCKS,),
        in_specs=[
            pl.BlockSpec((BB, H), lambda i: (i, 0)),
            pl.BlockSpec((L, BB, H), lambda i: (0, i, 0)),
            pl.BlockSpec((L, BB), lambda i: (0, i)),
            pl.BlockSpec((H, H), lambda i: (0, 0)),
            pl.BlockSpec((H, H), lambda i: (0, 0)),
            pl.BlockSpec((1, H), lambda i: (0, 0)),
        ],
        out_specs=pl.BlockSpec((PACK_ROWS, BB), lambda i: (i, 0)),
        out_shape=jax.ShapeDtypeStruct((TC_BLOCKS * PACK_ROWS, BB),
                                       jnp.float32),
    )(last_memory, all_t, mask_t, Wr, Ur, Vr)


VQ_STARTS = (0, 256, 512, 768)
VQ_ROWS = 256
VQ_LAST = V - VQ_STARTS[3]


def _sc_scatter_body(p_hbm, seq_hbm, out_hbm, p_v, seq_v, acc_v):
    wid = lax.axis_index("s") * NC + lax.axis_index("c")
    t = wid // 4
    q = wid % 4
    vstart = q * VQ_ROWS
    vend = jnp.minimum(vstart + VQ_ROWS, V)
    in_base = t * (BB * L)

    pltpu.sync_copy(p_hbm.at[pl.ds(t * PACK_ROWS, PACK_ROWS), :], p_v)
    pltpu.sync_copy(seq_hbm.at[pl.ds(in_base, BB * L)], seq_v)

    zeros16 = jnp.zeros((16,), jnp.float32)

    def _zero_rows(i, _):
        for u in range(8):
            acc_v[i, pl.ds(u * 16, 16)] = zeros16
        return 0

    lax.fori_loop(0, VQ_ROWS, _zero_rows, 0)

    lane = lax.iota(jnp.int32, 16)
    for g in range(BB // 16):
        b_idx = lane + g * 16
        lin = b_idx * L
        for l in range(L):
            col = plsc.load_gather(seq_v, [lin + l])
            val = p_v[l, pl.ds(g * 16, 16)]
            keep = (col >= vstart) & (col < vend)
            plsc.addupdate_scatter(acc_v, [col - vstart, b_idx],
                                   val, mask=keep)

    pltpu.sync_copy(acc_v.at[pl.ds(0, VQ_LAST), :],
                    out_hbm.at[pl.ds(vstart, VQ_LAST), pl.ds(t * BB, BB)])

    @pl.when(q < 3)
    def _tail():
        pltpu.sync_copy(
            acc_v.at[pl.ds(VQ_LAST, VQ_ROWS - VQ_LAST), :],
            out_hbm.at[pl.ds(vstart + VQ_LAST, VQ_ROWS - VQ_LAST),
                       pl.ds(t * BB, BB)])


@functools.cache
def _sc_scatter():
    return pl.kernel(
        _sc_scatter_body,
        out_type=jax.ShapeDtypeStruct((V, B), jnp.float32),
        mesh=plsc.VectorSubcoreMesh(core_axis_name="c", subcore_axis_name="s",
                                    num_cores=NC, num_subcores=NS),
        compiler_params=pltpu.CompilerParams(needs_layout_passes=False),
        scratch_types=[
            pltpu.VMEM((PACK_ROWS, BB), jnp.float32),
            pltpu.VMEM((BB * L,), jnp.int32),
            pltpu.VMEM((VQ_ROWS, BB), jnp.float32),
        ],
    )


def kernel(seq_item, last_memory, all_memory, mask, item_matrix, Wr, Ur, Vr):
    all_t = jnp.transpose(all_memory, (1, 0, 2))
    mask_t = mask.astype(jnp.int32).T
    probs = _tc_probs(last_memory, all_t, mask_t, Wr, Ur, Vr)
    out_t = _sc_scatter()(probs,
                          seq_item.astype(jnp.int32).reshape(B * L))
    return out_t.T


# speedup vs baseline: 1.3871x; 1.0629x over previous
import functools

import jax
import jax.numpy as jnp
from jax import lax
from jax.experimental import pallas as pl
from jax.experimental.pallas import tpu as pltpu
from jax.experimental.pallas import tpu_sc as plsc

B = 1024
L = 50
H = 128
V = 1000

NC = 2
NS = 16
NW = NC * NS
ROWS_PER_W = B // NW
GROUPS = ROWS_PER_W // 16

BB = 128
PACK_ROWS = 56
TC_BLOCKS = B // BB

MASK_SCORE = -60.0


def _probs_body(last_ref, all_ref, mask_ref, wr_ref, ur_ref, vr_ref, out_ref):
    lm = lax.dot_general(
        last_ref[...], wr_ref[...], (((1,), (1,)), ((), ())),
        preferred_element_type=jnp.float32)
    am = lax.dot_general(
        all_ref[...].reshape(L * BB, H), ur_ref[...],
        (((1,), (1,)), ((), ())),
        preferred_element_type=jnp.float32)
    z = jnp.tanh(am.reshape(L, BB, H) + lm[None, :, :])
    s = lax.dot_general(
        z.reshape(L * BB, H), vr_ref[...], (((1,), (1,)), ((), ())),
        preferred_element_type=jnp.float32)
    s2 = s.reshape(L, BB)
    s2 = jnp.where(mask_ref[...] != 0, MASK_SCORE, s2)
    e = jnp.exp(s2)
    p = e / jnp.sum(e, axis=0, keepdims=True)
    out_ref[...] = jnp.concatenate(
        [p, jnp.zeros((PACK_ROWS - L, BB), jnp.float32)], axis=0)


def _tc_probs(last_memory, all_t, mask_t, Wr, Ur, Vr):
    return pl.pallas_call(
        _probs_body,
        grid=(TC_BLOCKS,),
        in_specs=[
            pl.BlockSpec((BB, H), lambda i: (i, 0)),
            pl.BlockSpec((L, BB, H), lambda i: (0, i, 0)),
            pl.BlockSpec((L, BB), lambda i: (0, i)),
            pl.BlockSpec((H, H), lambda i: (0, 0)),
            pl.BlockSpec((H, H), lambda i: (0, 0)),
            pl.BlockSpec((1, H), lambda i: (0, 0)),
        ],
        out_specs=pl.BlockSpec((PACK_ROWS, BB), lambda i: (i, 0)),
        out_shape=jax.ShapeDtypeStruct((TC_BLOCKS * PACK_ROWS, BB),
                                       jnp.float32),
    )(last_memory, all_t, mask_t, Wr, Ur, Vr)


VQ_STARTS = (0, 256, 512, 768)
VQ_ROWS = 256
VQ_LAST = V - VQ_STARTS[3]


def _sc_scatter_body(p_hbm, seq_hbm, out_hbm, p_v, seq_v, acc_v):
    wid = lax.axis_index("s") * NC + lax.axis_index("c")
    t = wid // 4
    q = wid % 4
    vstart = q * VQ_ROWS
    vend = jnp.minimum(vstart + VQ_ROWS, V)

    pltpu.sync_copy(p_hbm.at[pl.ds(t * PACK_ROWS, PACK_ROWS), :], p_v)
    pltpu.sync_copy(seq_hbm.at[:, pl.ds(t * BB, BB)], seq_v)

    zeros16 = jnp.zeros((16,), jnp.float32)

    def _zero_rows(i, _):
        for u in range(8):
            acc_v[i, pl.ds(u * 16, 16)] = zeros16
        return 0

    lax.fori_loop(0, VQ_ROWS, _zero_rows, 0)

    lane = lax.iota(jnp.int32, 16)
    for g in range(BB // 16):
        b_idx = lane + g * 16
        for l in range(L):
            col = seq_v[l, pl.ds(g * 16, 16)]
            val = p_v[l, pl.ds(g * 16, 16)]
            keep = (col >= vstart) & (col < vend)
            plsc.addupdate_scatter(acc_v, [col - vstart, b_idx],
                                   val, mask=keep)

    pltpu.sync_copy(acc_v.at[pl.ds(0, VQ_LAST), :],
                    out_hbm.at[pl.ds(vstart, VQ_LAST), pl.ds(t * BB, BB)])

    @pl.when(q < 3)
    def _tail():
        pltpu.sync_copy(
            acc_v.at[pl.ds(VQ_LAST, VQ_ROWS - VQ_LAST), :],
            out_hbm.at[pl.ds(vstart + VQ_LAST, VQ_ROWS - VQ_LAST),
                       pl.ds(t * BB, BB)])


@functools.cache
def _sc_scatter():
    return pl.kernel(
        _sc_scatter_body,
        out_type=jax.ShapeDtypeStruct((V, B), jnp.float32),
        mesh=plsc.VectorSubcoreMesh(core_axis_name="c", subcore_axis_name="s",
                                    num_cores=NC, num_subcores=NS),
        compiler_params=pltpu.CompilerParams(needs_layout_passes=False),
        scratch_types=[
            pltpu.VMEM((PACK_ROWS, BB), jnp.float32),
            pltpu.VMEM((L, BB), jnp.int32),
            pltpu.VMEM((VQ_ROWS, BB), jnp.float32),
        ],
    )


def kernel(seq_item, last_memory, all_memory, mask, item_matrix, Wr, Ur, Vr):
    all_t = jnp.transpose(all_memory, (1, 0, 2))
    mask_t = mask.astype(jnp.int32).T
    probs = _tc_probs(last_memory, all_t, mask_t, Wr, Ur, Vr)
    out_t = _sc_scatter()(probs, seq_item.astype(jnp.int32).T)
    return out_t.T
